# baseline (speedup 1.0000x reference)
import functools

import jax
import jax.numpy as jnp
from jax import lax
from jax.experimental import pallas as pl
from jax.experimental.pallas import tpu as pltpu
from jax.experimental.pallas import tpu_sc as plsc

N = 10000
NPAD = 10240
E = 320000
EB = 64
ERT = 160
ECH = 32
EROWS = 5120
K = 10
ALPHA = 0.1
HID = 128
HIDW = HID // 2
NC = 2
NS = 16
ROWS_PER_TILE = NPAD // NS


@functools.lru_cache(maxsize=None)
def _mesh():
  return plsc.VectorSubcoreMesh(
      core_axis_name="c", subcore_axis_name="s", num_cores=NC,
      num_subcores=NS)


def _deg_body(dst_hbm, degp_out, deg_sh, degl, ones80, dstb):
  c = lax.axis_index("c")
  s = lax.axis_index("s")
  g = s * NC + c

  @pl.loop(0, ROWS_PER_TILE // 16)
  def _(i):
    degl[pl.ds(i * 16, 16)] = jnp.zeros((16,), jnp.float32)

  @pl.loop(0, EB // 16)
  def _(i):
    ones80[pl.ds(i * 16, 16)] = jnp.ones((16,), jnp.float32)

  pltpu.sync_copy(degl, deg_sh.at[pl.ds(s * ROWS_PER_TILE, ROWS_PER_TILE)])
  pltpu.sync_copy(dst_hbm.at[pl.ds(g * ERT, ERT)], dstb)
  plsc.subcore_barrier()

  @pl.loop(0, ERT)
  def _(b):
    pltpu.sync_copy(ones80, deg_sh.at[dstb.at[b]], add=True)

  plsc.subcore_barrier()
  pltpu.sync_copy(deg_sh.at[pl.ds(s * ROWS_PER_TILE, ROWS_PER_TILE)],
                  degp_out.at[c].at[pl.ds(s * ROWS_PER_TILE, ROWS_PER_TILE)])


@functools.lru_cache(maxsize=None)
def _deg_call():
  return pl.kernel(
      _deg_body,
      out_type=jax.ShapeDtypeStruct((NC, NPAD), jnp.float32),
      mesh=_mesh(),
      compiler_params=pltpu.CompilerParams(needs_layout_passes=False),
      scratch_types=[
          pltpu.VMEM_SHARED((NPAD,), jnp.float32),
          pltpu.VMEM((ROWS_PER_TILE,), jnp.float32),
          pltpu.VMEM((EB,), jnp.float32),
          pltpu.VMEM((ERT, EB), jnp.int32),
      ],
  )


def _step_body(y_hbm, src_hbm, dst_hbm, zeros_hbm, part_out,
               srcb, dstb, g0, g1, m0, m1, acc, gs0, gs1, ss0, ss1):
  c = lax.axis_index("c")
  s = lax.axis_index("s")
  g = s * NC + c

  pltpu.sync_copy(zeros_hbm.at[pl.ds(s * ROWS_PER_TILE, ROWS_PER_TILE)],
                  acc.at[pl.ds(s * ROWS_PER_TILE, ROWS_PER_TILE)])
  row0 = g * ERT
  plsc.subcore_barrier()

  gbufs = ((g0, gs0), (g1, gs1))
  mbufs = ((m0, ss0), (m1, ss1))
  himask = jnp.full((16,), -65536, jnp.int32)

  @pl.loop(0, ERT // ECH)
  def _(ch):
    pltpu.sync_copy(src_hbm.at[pl.ds(row0 + ch * ECH, ECH)], srcb)
    pltpu.sync_copy(dst_hbm.at[pl.ds(row0 + ch * ECH, ECH)], dstb)
    pltpu.async_copy(y_hbm.at[srcb.at[0]], g0, gs0)
    for b in range(ECH):
      gb, gs = gbufs[b % 2]
      mb, ss = mbufs[b % 2]
      if b + 1 < ECH:
        pltpu.async_copy(y_hbm.at[srcb.at[b + 1]], gbufs[(b + 1) % 2][0],
                         gbufs[(b + 1) % 2][1])
      pltpu.make_async_copy(y_hbm.at[srcb.at[b]], gb, gs).wait()
      if b >= 2:
        pltpu.make_async_copy(mb, acc.at[dstb.at[b - 2]], ss).wait()

      @pl.loop(0, EB)
      def _(e):
        for q in range(HID // 32):
          v = plsc.bitcast(gb[e, pl.ds(q * 32, 32)], jnp.int32)
          lo = lax.bitcast_convert_type(
              lax.shift_left(v, jnp.full((16,), 16, jnp.int32)), jnp.float32)
          hi = lax.bitcast_convert_type(v & himask, jnp.float32)
          mb[e, pl.ds(q * 32, 16)] = lo
          mb[e, pl.ds(q * 32 + 16, 16)] = hi

      pltpu.async_copy(mb, acc.at[dstb.at[b]], ss, add=True)
    pltpu.make_async_copy(m0, acc.at[dstb.at[ECH - 2]], ss0).wait()
    pltpu.make_async_copy(m1, acc.at[dstb.at[ECH - 1]], ss1).wait()

  plsc.subcore_barrier()
  pltpu.sync_copy(acc.at[pl.ds(s * ROWS_PER_TILE, ROWS_PER_TILE)],
                  part_out.at[c].at[pl.ds(s * ROWS_PER_TILE, ROWS_PER_TILE)])


@functools.lru_cache(maxsize=None)
def _step_call():
  return pl.kernel(
      _step_body,
      out_type=jax.ShapeDtypeStruct((NC, NPAD, HID), jnp.float32),
      mesh=_mesh(),
      compiler_params=pltpu.CompilerParams(needs_layout_passes=False,
                                           use_tc_tiling_on_sc=False),
      scratch_types=[
          pltpu.VMEM((ECH, EB), jnp.int32),
          pltpu.VMEM((ECH, EB), jnp.int32),
          pltpu.VMEM((EB, HID), jnp.bfloat16),
          pltpu.VMEM((EB, HID), jnp.bfloat16),
          pltpu.VMEM((EB, HID), jnp.float32),
          pltpu.VMEM((EB, HID), jnp.float32),
          pltpu.VMEM_SHARED((NPAD, HID), jnp.float32),
          pltpu.SemaphoreType.DMA,
          pltpu.SemaphoreType.DMA,
          pltpu.SemaphoreType.DMA,
          pltpu.SemaphoreType.DMA,
      ],
  )


_BLK = 1024


def _pack_y(y):
  b = y.shape[0]
  yp = y.reshape(b, HID // 32, 2, 16).swapaxes(2, 3).reshape(b, HID)
  return yp.astype(jnp.bfloat16)


def _lin1_body(x_ref, w_ref, b_ref, pd0_ref, pd1_ref,
               z0_ref, y0_ref, dinv_ref):
  i = pl.program_id(0)
  rows = lax.broadcasted_iota(jnp.int32, (_BLK, 1), 0) + i * _BLK
  real = rows < N
  deg = pd0_ref[...] + pd1_ref[...] + 1.0
  dinv = jnp.where(real, lax.rsqrt(deg), 0.0)
  h = jnp.dot(x_ref[...], w_ref[...],
              preferred_element_type=jnp.float32) + b_ref[...]
  h = jnp.where(real, h, 0.0)
  z0_ref[...] = h
  y0_ref[...] = _pack_y(dinv * h)
  dinv_ref[...] = dinv


def _lin1(x_pad, W1, b1, pd0, pd1):
  return pl.pallas_call(
      _lin1_body,
      grid=(NPAD // _BLK,),
      in_specs=[pl.BlockSpec((_BLK, HID), lambda i: (i, 0)),
                pl.BlockSpec((HID, HID), lambda i: (0, 0)),
                pl.BlockSpec((1, HID), lambda i: (0, 0)),
                pl.BlockSpec((_BLK, 1), lambda i: (i, 0)),
                pl.BlockSpec((_BLK, 1), lambda i: (i, 0))],
      out_specs=[pl.BlockSpec((_BLK, HID), lambda i: (i, 0)),
                 pl.BlockSpec((_BLK, HID), lambda i: (i, 0)),
                 pl.BlockSpec((_BLK, 1), lambda i: (i, 0))],
      out_shape=[jax.ShapeDtypeStruct((NPAD, HID), jnp.float32),
                 jax.ShapeDtypeStruct((NPAD, HID), jnp.bfloat16),
                 jax.ShapeDtypeStruct((NPAD, 1), jnp.float32)],
  )(x_pad, W1, b1, pd0, pd1)


def _comb_body(p0_ref, p1_ref, dv_ref, z_ref, h_ref, z1_ref, y1_ref):
  dv = dv_ref[...]
  z1 = ((1.0 - ALPHA) * (dv * (p0_ref[...] + p1_ref[...])
                         + dv * dv * z_ref[...])
        + ALPHA * h_ref[...])
  z1_ref[...] = z1
  y1_ref[...] = _pack_y(dv * z1)


def _combine(p0, p1, dinvc, z, h):
  return pl.pallas_call(
      _comb_body,
      grid=(NPAD // _BLK,),
      in_specs=[pl.BlockSpec((_BLK, HID), lambda i: (i, 0)),
                pl.BlockSpec((_BLK, HID), lambda i: (i, 0)),
                pl.BlockSpec((_BLK, 1), lambda i: (i, 0)),
                pl.BlockSpec((_BLK, HID), lambda i: (i, 0)),
                pl.BlockSpec((_BLK, HID), lambda i: (i, 0))],
      out_specs=[pl.BlockSpec((_BLK, HID), lambda i: (i, 0)),
                 pl.BlockSpec((_BLK, HID), lambda i: (i, 0))],
      out_shape=[jax.ShapeDtypeStruct((NPAD, HID), jnp.float32),
                 jax.ShapeDtypeStruct((NPAD, HID), jnp.bfloat16)],
  )(p0, p1, dinvc, z, h)


def _stats_body(z_ref, ms_ref):
  z = z_ref[...]
  mu = jnp.sum(z, axis=0, keepdims=True) / N
  var = jnp.sum(z * z, axis=0, keepdims=True) / N - mu * mu
  rstd = lax.rsqrt(var + 1e-5)
  ms_ref[...] = jnp.concatenate([mu, rstd], axis=0)


def _stats(z):
  return pl.pallas_call(
      _stats_body,
      out_shape=jax.ShapeDtypeStruct((2, HID), jnp.float32),
  )(z)


def _final_body(z_ref, ms_ref, g_ref, be_ref, w2_ref, b2_ref,
                logp_ref, logits_ref):
  zn = ((z_ref[...] - ms_ref[0:1, :]) * ms_ref[1:2, :] * g_ref[...]
        + be_ref[...])
  lg = jnp.dot(zn, w2_ref[...],
               preferred_element_type=jnp.float32) + b2_ref[...]
  m = jnp.max(lg, axis=1, keepdims=True)
  lse = jnp.log(jnp.sum(jnp.exp(lg - m), axis=1, keepdims=True)) + m
  logits_ref[...] = lg
  logp_ref[...] = lg - lse


def _final(z, ms, gamma, beta, W2, b2):
  odim = W2.shape[1]
  return pl.pallas_call(
      _final_body,
      grid=(NPAD // _BLK,),
      in_specs=[pl.BlockSpec((_BLK, HID), lambda i: (i, 0)),
                pl.BlockSpec((2, HID), lambda i: (0, 0)),
                pl.BlockSpec((1, HID), lambda i: (0, 0)),
                pl.BlockSpec((1, HID), lambda i: (0, 0)),
                pl.BlockSpec((HID, odim), lambda i: (0, 0)),
                pl.BlockSpec((1, odim), lambda i: (0, 0))],
      out_specs=[pl.BlockSpec((_BLK, odim), lambda i: (i, 0)),
                 pl.BlockSpec((_BLK, odim), lambda i: (i, 0))],
      out_shape=[jax.ShapeDtypeStruct((NPAD, odim), jnp.float32),
                 jax.ShapeDtypeStruct((NPAD, odim), jnp.float32)],
  )(z, ms, gamma, beta, W2, b2)


def kernel(x, edge_index, W1, b1, gamma, beta, W2, b2):
  x_pad = jnp.pad(x, ((0, NPAD - N), (0, 0)))
  ntiles = NC * NS
  per_tile = E // ntiles
  pad_n = ERT * EB - per_tile

  def pad_edges(v):
    v2 = v.reshape(ntiles, per_tile)
    fill = jnp.full((ntiles, pad_n), N, jnp.int32)
    return jnp.concatenate([v2, fill], axis=1).reshape(EROWS, EB)

  src2 = pad_edges(edge_index[0])
  dst2 = pad_edges(edge_index[1])
  zeros = jnp.zeros((NPAD, HID), jnp.float32)

  degp = _deg_call()(dst2)
  z0, y0, dinvc = _lin1(x_pad, W1, b1.reshape(1, HID),
                        degp[0].reshape(NPAD, 1), degp[1].reshape(NPAD, 1))

  def one_step(_, carry):
    z, y = carry
    parts = _step_call()(y, src2, dst2, zeros)
    return _combine(parts[0], parts[1], dinvc, z, z0)

  z, _ = lax.fori_loop(0, K, one_step, (z0, y0))

  ms = _stats(z)
  logp, logits = _final(z, ms, gamma.reshape(1, HID), beta.reshape(1, HID),
                        W2, b2.reshape(1, W2.shape[1]))
  return (logp[:N], logits[:N])


# speedup vs baseline: 1.2804x; 1.2804x over previous
import functools

import jax
import jax.numpy as jnp
from jax import lax
from jax.experimental import pallas as pl
from jax.experimental.pallas import tpu as pltpu
from jax.experimental.pallas import tpu_sc as plsc

N = 10000
NPAD = 10240
E = 320000
EB = 80
ERT = 128
ECH = 32
EROWS = 4096
K = 10
ALPHA = 0.1
HID = 128
NC = 2
NS = 16
ROWS_PER_TILE = NPAD // NS


@functools.lru_cache(maxsize=None)
def _mesh():
  return plsc.VectorSubcoreMesh(
      core_axis_name="c", subcore_axis_name="s", num_cores=NC,
      num_subcores=NS)


def _deg_body(dst_hbm, degp_out, deg_sh, degl, ones80, dstb, dsem):
  c = lax.axis_index("c")
  s = lax.axis_index("s")
  g = s * NC + c

  @pl.loop(0, ROWS_PER_TILE // 16)
  def _(i):
    degl[pl.ds(i * 16, 16)] = jnp.zeros((16,), jnp.float32)

  @pl.loop(0, EB // 16)
  def _(i):
    ones80[pl.ds(i * 16, 16)] = jnp.ones((16,), jnp.float32)

  pltpu.sync_copy(degl, deg_sh.at[pl.ds(s * ROWS_PER_TILE, ROWS_PER_TILE)])
  pltpu.sync_copy(dst_hbm.at[pl.ds(g * ERT, ERT)], dstb)
  plsc.subcore_barrier()

  @pl.loop(0, ERT // 8)
  def _(i):
    for j in range(8):
      pltpu.async_copy(ones80, deg_sh.at[dstb.at[i * 8 + j]], dsem, add=True)
    for j in range(8):
      pltpu.make_async_copy(ones80, deg_sh.at[dstb.at[i * 8 + j]],
                            dsem).wait()

  plsc.subcore_barrier()
  pltpu.sync_copy(deg_sh.at[pl.ds(s * ROWS_PER_TILE, ROWS_PER_TILE)],
                  degp_out.at[c].at[pl.ds(s * ROWS_PER_TILE, ROWS_PER_TILE)])


@functools.lru_cache(maxsize=None)
def _deg_call():
  return pl.kernel(
      _deg_body,
      out_type=jax.ShapeDtypeStruct((NC, NPAD), jnp.float32),
      mesh=_mesh(),
      compiler_params=pltpu.CompilerParams(needs_layout_passes=False),
      scratch_types=[
          pltpu.VMEM_SHARED((NPAD,), jnp.float32),
          pltpu.VMEM((ROWS_PER_TILE,), jnp.float32),
          pltpu.VMEM((EB,), jnp.float32),
          pltpu.VMEM((ERT, EB), jnp.int32),
          pltpu.SemaphoreType.DMA,
      ],
  )


def _step_body(y_hbm, src_hbm, dst_hbm, zeros_hbm, part_out,
               srcb, dstb, m0, m1, acc, gs0, gs1, ss0, ss1):
  c = lax.axis_index("c")
  s = lax.axis_index("s")
  g = s * NC + c

  pltpu.sync_copy(zeros_hbm.at[pl.ds(s * ROWS_PER_TILE, ROWS_PER_TILE)],
                  acc.at[pl.ds(s * ROWS_PER_TILE, ROWS_PER_TILE)])
  row0 = g * ERT
  plsc.subcore_barrier()

  bufs = ((m0, gs0, ss0), (m1, gs1, ss1))

  @pl.loop(0, ERT // ECH)
  def _(ch):
    pltpu.sync_copy(src_hbm.at[pl.ds(row0 + ch * ECH, ECH)], srcb)
    pltpu.sync_copy(dst_hbm.at[pl.ds(row0 + ch * ECH, ECH)], dstb)
    pltpu.async_copy(y_hbm.at[srcb.at[0]], m0, gs0)
    for b in range(ECH):
      mb, gs, ss = bufs[b % 2]
      nb, gn, sn = bufs[(b + 1) % 2]
      if b + 1 < ECH:
        if b >= 1:
          pltpu.make_async_copy(nb, acc.at[dstb.at[b - 1]], sn).wait()
        pltpu.async_copy(y_hbm.at[srcb.at[b + 1]], nb, gn)
      pltpu.make_async_copy(y_hbm.at[srcb.at[b]], mb, gs).wait()
      pltpu.async_copy(mb, acc.at[dstb.at[b]], ss, add=True)
    pltpu.make_async_copy(m0, acc.at[dstb.at[ECH - 2]], ss0).wait()
    pltpu.make_async_copy(m1, acc.at[dstb.at[ECH - 1]], ss1).wait()

  plsc.subcore_barrier()
  pltpu.sync_copy(acc.at[pl.ds(s * ROWS_PER_TILE, ROWS_PER_TILE)],
                  part_out.at[c].at[pl.ds(s * ROWS_PER_TILE, ROWS_PER_TILE)])


@functools.lru_cache(maxsize=None)
def _step_call():
  return pl.kernel(
      _step_body,
      out_type=jax.ShapeDtypeStruct((NC, NPAD, HID), jnp.float32),
      mesh=_mesh(),
      compiler_params=pltpu.CompilerParams(needs_layout_passes=False),
      scratch_types=[
          pltpu.VMEM((ECH, EB), jnp.int32),
          pltpu.VMEM((ECH, EB), jnp.int32),
          pltpu.VMEM((EB, HID), jnp.float32),
          pltpu.VMEM((EB, HID), jnp.float32),
          pltpu.VMEM_SHARED((NPAD, HID), jnp.float32),
          pltpu.SemaphoreType.DMA,
          pltpu.SemaphoreType.DMA,
          pltpu.SemaphoreType.DMA,
          pltpu.SemaphoreType.DMA,
      ],
  )


_BLK = 1024


def _lin1_body(x_ref, w_ref, b_ref, pd0_ref, pd1_ref,
               z0_ref, y0_ref, dinv_ref):
  i = pl.program_id(0)
  rows = lax.broadcasted_iota(jnp.int32, (_BLK, 1), 0) + i * _BLK
  real = rows < N
  deg = pd0_ref[...] + pd1_ref[...] + 1.0
  dinv = jnp.where(real, lax.rsqrt(deg), 0.0)
  h = jnp.dot(x_ref[...], w_ref[...],
              preferred_element_type=jnp.float32) + b_ref[...]
  h = jnp.where(real, h, 0.0)
  z0_ref[...] = h
  y0_ref[...] = dinv * h
  dinv_ref[...] = dinv


def _lin1(x_pad, W1, b1, pd0, pd1):
  return pl.pallas_call(
      _lin1_body,
      grid=(NPAD // _BLK,),
      in_specs=[pl.BlockSpec((_BLK, HID), lambda i: (i, 0)),
                pl.BlockSpec((HID, HID), lambda i: (0, 0)),
                pl.BlockSpec((1, HID), lambda i: (0, 0)),
                pl.BlockSpec((_BLK, 1), lambda i: (i, 0)),
                pl.BlockSpec((_BLK, 1), lambda i: (i, 0))],
      out_specs=[pl.BlockSpec((_BLK, HID), lambda i: (i, 0)),
                 pl.BlockSpec((_BLK, HID), lambda i: (i, 0)),
                 pl.BlockSpec((_BLK, 1), lambda i: (i, 0))],
      out_shape=[jax.ShapeDtypeStruct((NPAD, HID), jnp.float32),
                 jax.ShapeDtypeStruct((NPAD, HID), jnp.float32),
                 jax.ShapeDtypeStruct((NPAD, 1), jnp.float32)],
  )(x_pad, W1, b1, pd0, pd1)


def _comb_body(p0_ref, p1_ref, dv_ref, z_ref, h_ref, z1_ref, y1_ref):
  dv = dv_ref[...]
  z1 = ((1.0 - ALPHA) * (dv * (p0_ref[...] + p1_ref[...])
                         + dv * dv * z_ref[...])
        + ALPHA * h_ref[...])
  z1_ref[...] = z1
  y1_ref[...] = dv * z1


def _combine(p0, p1, dinvc, z, h):
  return pl.pallas_call(
      _comb_body,
      grid=(NPAD // _BLK,),
      in_specs=[pl.BlockSpec((_BLK, HID), lambda i: (i, 0)),
                pl.BlockSpec((_BLK, HID), lambda i: (i, 0)),
                pl.BlockSpec((_BLK, 1), lambda i: (i, 0)),
                pl.BlockSpec((_BLK, HID), lambda i: (i, 0)),
                pl.BlockSpec((_BLK, HID), lambda i: (i, 0))],
      out_specs=[pl.BlockSpec((_BLK, HID), lambda i: (i, 0)),
                 pl.BlockSpec((_BLK, HID), lambda i: (i, 0))],
      out_shape=[jax.ShapeDtypeStruct((NPAD, HID), jnp.float32),
                 jax.ShapeDtypeStruct((NPAD, HID), jnp.float32)],
  )(p0, p1, dinvc, z, h)


def _stats_body(z_ref, ms_ref):
  z = z_ref[...]
  mu = jnp.sum(z, axis=0, keepdims=True) / N
  var = jnp.sum(z * z, axis=0, keepdims=True) / N - mu * mu
  rstd = lax.rsqrt(var + 1e-5)
  ms_ref[...] = jnp.concatenate([mu, rstd], axis=0)


def _stats(z):
  return pl.pallas_call(
      _stats_body,
      out_shape=jax.ShapeDtypeStruct((2, HID), jnp.float32),
  )(z)


def _final_body(z_ref, ms_ref, g_ref, be_ref, w2_ref, b2_ref,
                logp_ref, logits_ref):
  zn = ((z_ref[...] - ms_ref[0:1, :]) * ms_ref[1:2, :] * g_ref[...]
        + be_ref[...])
  lg = jnp.dot(zn, w2_ref[...],
               preferred_element_type=jnp.float32) + b2_ref[...]
  m = jnp.max(lg, axis=1, keepdims=True)
  lse = jnp.log(jnp.sum(jnp.exp(lg - m), axis=1, keepdims=True)) + m
  logits_ref[...] = lg
  logp_ref[...] = lg - lse


def _final(z, ms, gamma, beta, W2, b2):
  odim = W2.shape[1]
  return pl.pallas_call(
      _final_body,
      grid=(NPAD // _BLK,),
      in_specs=[pl.BlockSpec((_BLK, HID), lambda i: (i, 0)),
                pl.BlockSpec((2, HID), lambda i: (0, 0)),
                pl.BlockSpec((1, HID), lambda i: (0, 0)),
                pl.BlockSpec((1, HID), lambda i: (0, 0)),
                pl.BlockSpec((HID, odim), lambda i: (0, 0)),
                pl.BlockSpec((1, odim), lambda i: (0, 0))],
      out_specs=[pl.BlockSpec((_BLK, odim), lambda i: (i, 0)),
                 pl.BlockSpec((_BLK, odim), lambda i: (i, 0))],
      out_shape=[jax.ShapeDtypeStruct((NPAD, odim), jnp.float32),
                 jax.ShapeDtypeStruct((NPAD, odim), jnp.float32)],
  )(z, ms, gamma, beta, W2, b2)


def kernel(x, edge_index, W1, b1, gamma, beta, W2, b2):
  x_pad = jnp.pad(x, ((0, NPAD - N), (0, 0)))
  ntiles = NC * NS
  real_rows = E // EB // ntiles
  pad_rows = ERT - real_rows

  def pad_edges(v):
    v3 = v.reshape(ntiles, real_rows, EB)
    fill = jnp.full((ntiles, pad_rows, EB), N, jnp.int32)
    return jnp.concatenate([v3, fill], axis=1).reshape(EROWS, EB)

  src2 = pad_edges(edge_index[0])
  dst2 = pad_edges(edge_index[1])
  zeros = jnp.zeros((NPAD, HID), jnp.float32)

  degp = _deg_call()(dst2)
  z0, y0, dinvc = _lin1(x_pad, W1, b1.reshape(1, HID),
                        degp[0].reshape(NPAD, 1), degp[1].reshape(NPAD, 1))

  def one_step(_, carry):
    z, y = carry
    parts = _step_call()(y, src2, dst2, zeros)
    return _combine(parts[0], parts[1], dinvc, z, z0)

  z, _ = lax.fori_loop(0, K, one_step, (z0, y0))

  ms = _stats(z)
  logp, logits = _final(z, ms, gamma.reshape(1, HID), beta.reshape(1, HID),
                        W2, b2.reshape(1, W2.shape[1]))
  return (logp[:N], logits[:N])


# speedup vs baseline: 1.2927x; 1.0096x over previous
import functools

import jax
import jax.numpy as jnp
from jax import lax
from jax.experimental import pallas as pl
from jax.experimental.pallas import tpu as pltpu
from jax.experimental.pallas import tpu_sc as plsc

N = 10000
NPAD = 10240
E = 320000
EB = 80
ERT = 128
ECH = 64
EROWS = 4096
K = 10
ALPHA = 0.1
HID = 128
NC = 2
NS = 16
ROWS_PER_TILE = NPAD // NS


@functools.lru_cache(maxsize=None)
def _mesh():
  return plsc.VectorSubcoreMesh(
      core_axis_name="c", subcore_axis_name="s", num_cores=NC,
      num_subcores=NS)


def _deg_body(dst_hbm, degp_out, deg_sh, degl, ones80, dstb, dsem):
  c = lax.axis_index("c")
  s = lax.axis_index("s")
  g = s * NC + c

  @pl.loop(0, ROWS_PER_TILE // 16)
  def _(i):
    degl[pl.ds(i * 16, 16)] = jnp.zeros((16,), jnp.float32)

  @pl.loop(0, EB // 16)
  def _(i):
    ones80[pl.ds(i * 16, 16)] = jnp.ones((16,), jnp.float32)

  pltpu.sync_copy(degl, deg_sh.at[pl.ds(s * ROWS_PER_TILE, ROWS_PER_TILE)])
  pltpu.sync_copy(dst_hbm.at[pl.ds(g * ERT, ERT)], dstb)
  plsc.subcore_barrier()

  @pl.loop(0, ERT // 8)
  def _(i):
    for j in range(8):
      pltpu.async_copy(ones80, deg_sh.at[dstb.at[i * 8 + j]], dsem, add=True)
    for j in range(8):
      pltpu.make_async_copy(ones80, deg_sh.at[dstb.at[i * 8 + j]],
                            dsem).wait()

  plsc.subcore_barrier()
  pltpu.sync_copy(deg_sh.at[pl.ds(s * ROWS_PER_TILE, ROWS_PER_TILE)],
                  degp_out.at[c].at[pl.ds(s * ROWS_PER_TILE, ROWS_PER_TILE)])


@functools.lru_cache(maxsize=None)
def _deg_call():
  return pl.kernel(
      _deg_body,
      out_type=jax.ShapeDtypeStruct((NC, NPAD), jnp.float32),
      mesh=_mesh(),
      compiler_params=pltpu.CompilerParams(needs_layout_passes=False),
      scratch_types=[
          pltpu.VMEM_SHARED((NPAD,), jnp.float32),
          pltpu.VMEM((ROWS_PER_TILE,), jnp.float32),
          pltpu.VMEM((EB,), jnp.float32),
          pltpu.VMEM((ERT, EB), jnp.int32),
          pltpu.SemaphoreType.DMA,
      ],
  )


def _step_body(y_hbm, src_hbm, dst_hbm, zeros_hbm, part_out,
               srcb, dstb, m0, m1, acc, gs0, gs1, ss0, ss1):
  c = lax.axis_index("c")
  s = lax.axis_index("s")
  g = s * NC + c

  pltpu.sync_copy(zeros_hbm.at[pl.ds(s * ROWS_PER_TILE, ROWS_PER_TILE)],
                  acc.at[pl.ds(s * ROWS_PER_TILE, ROWS_PER_TILE)])
  row0 = g * ERT
  plsc.subcore_barrier()

  bufs = ((m0, gs0, ss0), (m1, gs1, ss1))

  @pl.loop(0, ERT // ECH)
  def _(ch):
    pltpu.sync_copy(src_hbm.at[pl.ds(row0 + ch * ECH, ECH)], srcb)
    pltpu.sync_copy(dst_hbm.at[pl.ds(row0 + ch * ECH, ECH)], dstb)
    pltpu.async_copy(y_hbm.at[srcb.at[0]], m0, gs0)
    for b in range(ECH):
      mb, gs, ss = bufs[b % 2]
      nb, gn, sn = bufs[(b + 1) % 2]
      if b + 1 < ECH:
        if b >= 1:
          pltpu.make_async_copy(nb, acc.at[dstb.at[b - 1]], sn).wait()
        pltpu.async_copy(y_hbm.at[srcb.at[b + 1]], nb, gn)
      pltpu.make_async_copy(y_hbm.at[srcb.at[b]], mb, gs).wait()
      pltpu.async_copy(mb, acc.at[dstb.at[b]], ss, add=True)
    pltpu.make_async_copy(m0, acc.at[dstb.at[ECH - 2]], ss0).wait()
    pltpu.make_async_copy(m1, acc.at[dstb.at[ECH - 1]], ss1).wait()

  plsc.subcore_barrier()
  pltpu.sync_copy(acc.at[pl.ds(s * ROWS_PER_TILE, ROWS_PER_TILE)],
                  part_out.at[c].at[pl.ds(s * ROWS_PER_TILE, ROWS_PER_TILE)])


@functools.lru_cache(maxsize=None)
def _step_call():
  return pl.kernel(
      _step_body,
      out_type=jax.ShapeDtypeStruct((NC, NPAD, HID), jnp.float32),
      mesh=_mesh(),
      compiler_params=pltpu.CompilerParams(needs_layout_passes=False),
      scratch_types=[
          pltpu.VMEM((ECH, EB), jnp.int32),
          pltpu.VMEM((ECH, EB), jnp.int32),
          pltpu.VMEM((EB, HID), jnp.float32),
          pltpu.VMEM((EB, HID), jnp.float32),
          pltpu.VMEM_SHARED((NPAD, HID), jnp.float32),
          pltpu.SemaphoreType.DMA,
          pltpu.SemaphoreType.DMA,
          pltpu.SemaphoreType.DMA,
          pltpu.SemaphoreType.DMA,
      ],
  )


_BLK = 1024


def _lin1_body(x_ref, w_ref, b_ref, pd0_ref, pd1_ref,
               z0_ref, y0_ref, dinv_ref):
  i = pl.program_id(0)
  rows = lax.broadcasted_iota(jnp.int32, (_BLK, 1), 0) + i * _BLK
  real = rows < N
  deg = pd0_ref[...] + pd1_ref[...] + 1.0
  dinv = jnp.where(real, lax.rsqrt(deg), 0.0)
  h = jnp.dot(x_ref[...], w_ref[...],
              preferred_element_type=jnp.float32) + b_ref[...]
  h = jnp.where(real, h, 0.0)
  z0_ref[...] = h
  y0_ref[...] = dinv * h
  dinv_ref[...] = dinv


def _lin1(x_pad, W1, b1, pd0, pd1):
  return pl.pallas_call(
      _lin1_body,
      grid=(NPAD // _BLK,),
      in_specs=[pl.BlockSpec((_BLK, HID), lambda i: (i, 0)),
                pl.BlockSpec((HID, HID), lambda i: (0, 0)),
                pl.BlockSpec((1, HID), lambda i: (0, 0)),
                pl.BlockSpec((_BLK, 1), lambda i: (i, 0)),
                pl.BlockSpec((_BLK, 1), lambda i: (i, 0))],
      out_specs=[pl.BlockSpec((_BLK, HID), lambda i: (i, 0)),
                 pl.BlockSpec((_BLK, HID), lambda i: (i, 0)),
                 pl.BlockSpec((_BLK, 1), lambda i: (i, 0))],
      out_shape=[jax.ShapeDtypeStruct((NPAD, HID), jnp.float32),
                 jax.ShapeDtypeStruct((NPAD, HID), jnp.float32),
                 jax.ShapeDtypeStruct((NPAD, 1), jnp.float32)],
  )(x_pad, W1, b1, pd0, pd1)


def _comb_body(p0_ref, p1_ref, dv_ref, z_ref, h_ref, z1_ref, y1_ref):
  dv = dv_ref[...]
  z1 = ((1.0 - ALPHA) * (dv * (p0_ref[...] + p1_ref[...])
                         + dv * dv * z_ref[...])
        + ALPHA * h_ref[...])
  z1_ref[...] = z1
  y1_ref[...] = dv * z1


def _combine(p0, p1, dinvc, z, h):
  return pl.pallas_call(
      _comb_body,
      grid=(NPAD // _BLK,),
      in_specs=[pl.BlockSpec((_BLK, HID), lambda i: (i, 0)),
                pl.BlockSpec((_BLK, HID), lambda i: (i, 0)),
                pl.BlockSpec((_BLK, 1), lambda i: (i, 0)),
                pl.BlockSpec((_BLK, HID), lambda i: (i, 0)),
                pl.BlockSpec((_BLK, HID), lambda i: (i, 0))],
      out_specs=[pl.BlockSpec((_BLK, HID), lambda i: (i, 0)),
                 pl.BlockSpec((_BLK, HID), lambda i: (i, 0))],
      out_shape=[jax.ShapeDtypeStruct((NPAD, HID), jnp.float32),
                 jax.ShapeDtypeStruct((NPAD, HID), jnp.float32)],
  )(p0, p1, dinvc, z, h)


def _stats_body(z_ref, ms_ref):
  z = z_ref[...]
  mu = jnp.sum(z, axis=0, keepdims=True) / N
  var = jnp.sum(z * z, axis=0, keepdims=True) / N - mu * mu
  rstd = lax.rsqrt(var + 1e-5)
  ms_ref[...] = jnp.concatenate([mu, rstd], axis=0)


def _stats(z):
  return pl.pallas_call(
      _stats_body,
      out_shape=jax.ShapeDtypeStruct((2, HID), jnp.float32),
  )(z)


def _final_body(z_ref, ms_ref, g_ref, be_ref, w2_ref, b2_ref,
                logp_ref, logits_ref):
  zn = ((z_ref[...] - ms_ref[0:1, :]) * ms_ref[1:2, :] * g_ref[...]
        + be_ref[...])
  lg = jnp.dot(zn, w2_ref[...],
               preferred_element_type=jnp.float32) + b2_ref[...]
  m = jnp.max(lg, axis=1, keepdims=True)
  lse = jnp.log(jnp.sum(jnp.exp(lg - m), axis=1, keepdims=True)) + m
  logits_ref[...] = lg
  logp_ref[...] = lg - lse


def _final(z, ms, gamma, beta, W2, b2):
  odim = W2.shape[1]
  return pl.pallas_call(
      _final_body,
      grid=(NPAD // _BLK,),
      in_specs=[pl.BlockSpec((_BLK, HID), lambda i: (i, 0)),
                pl.BlockSpec((2, HID), lambda i: (0, 0)),
                pl.BlockSpec((1, HID), lambda i: (0, 0)),
                pl.BlockSpec((1, HID), lambda i: (0, 0)),
                pl.BlockSpec((HID, odim), lambda i: (0, 0)),
                pl.BlockSpec((1, odim), lambda i: (0, 0))],
      out_specs=[pl.BlockSpec((_BLK, odim), lambda i: (i, 0)),
                 pl.BlockSpec((_BLK, odim), lambda i: (i, 0))],
      out_shape=[jax.ShapeDtypeStruct((NPAD, odim), jnp.float32),
                 jax.ShapeDtypeStruct((NPAD, odim), jnp.float32)],
  )(z, ms, gamma, beta, W2, b2)


def kernel(x, edge_index, W1, b1, gamma, beta, W2, b2):
  x_pad = jnp.pad(x, ((0, NPAD - N), (0, 0)))
  ntiles = NC * NS
  real_rows = E // EB // ntiles
  pad_rows = ERT - real_rows

  def pad_edges(v):
    v3 = v.reshape(ntiles, real_rows, EB)
    fill = jnp.full((ntiles, pad_rows, EB), N, jnp.int32)
    return jnp.concatenate([v3, fill], axis=1).reshape(EROWS, EB)

  src2 = pad_edges(edge_index[0])
  dst2 = pad_edges(edge_index[1])
  zeros = jnp.zeros((NPAD, HID), jnp.float32)

  degp = _deg_call()(dst2)
  z0, y0, dinvc = _lin1(x_pad, W1, b1.reshape(1, HID),
                        degp[0].reshape(NPAD, 1), degp[1].reshape(NPAD, 1))

  def one_step(_, carry):
    z, y = carry
    parts = _step_call()(y, src2, dst2, zeros)
    return _combine(parts[0], parts[1], dinvc, z, z0)

  z, _ = lax.fori_loop(0, K, one_step, (z0, y0))

  ms = _stats(z)
  logp, logits = _final(z, ms, gamma.reshape(1, HID), beta.reshape(1, HID),
                        W2, b2.reshape(1, W2.shape[1]))
  return (logp[:N], logits[:N])
